# baseline (speedup 1.0000x reference)
import functools

import jax
import jax.numpy as jnp
from jax import lax
from jax.experimental import pallas as pl
from jax.experimental.pallas import tpu as pltpu
from jax.experimental.pallas import tpu_sc as plsc

_SENT = (1 << 14) - 1
_NC = 2
_NS = 16
_SP = 512


def _sc_gather(ef_hbm, data_hbm, g_hbm, valid_hbm, srcs_hbm,
               sd_v, s16_v, shift_v, idx_v, val_v, rows_v, sem,
               *, e, n):
    wid = lax.axis_index("s") * _NC + lax.axis_index("c")
    base = wid * 64
    ntile = (4 * _SP) // 64
    lane = jnp.arange(16, dtype=jnp.int32)
    row0 = jnp.zeros((16,), jnp.int32)
    row1 = jnp.full((16,), 1, jnp.int32)

    pltpu.sync_copy(ef_hbm.at[pl.ds(base, 64)], sd_v.at[0])

    @pl.when(wid < ntile - 1)
    def _():
        pltpu.sync_copy(ef_hbm.at[pl.ds(e + base, 64)], sd_v.at[1])

    @pl.when(wid == ntile - 1)
    def _():
        pltpu.sync_copy(ef_hbm.at[pl.ds(e + base, 16)],
                        sd_v.at[1, pl.ds(0, 16)])
        for v in range(1, 4):
            plsc.store_scatter(sd_v, [row1, 16 * v + lane],
                               jnp.full((16,), _SENT, jnp.int32))

    grp = (lane >> 2) << 14
    for v in range(4):
        d = plsc.load_gather(sd_v, [row1, 16 * v + lane])
        key = grp | d
        k1, _ = plsc.sort_key_val(key, lane)
        shift_v[...] = k1
        prev = plsc.load_gather(shift_v, [jnp.maximum(lane - 1, 0)])
        dup = (k1 == prev) & (lane != 0)
        k2 = jnp.where(dup, grp | _SENT, k1)
        k3, _ = plsc.sort_key_val(k2, lane)
        dstf = k3 & _SENT
        validb = dstf != _SENT
        posv = ((lane & 3) << 4) + 4 * v + (lane >> 2)
        plsc.store_scatter(idx_v, [posv], jnp.where(validb, dstf, 0))
        plsc.store_scatter(val_v, [4 * v + (lane >> 2), lane & 3],
                           validb.astype(jnp.float32))

    s16 = plsc.load_gather(sd_v, [row0, lane * 4])
    pad = (wid * 16 + lane) >= (e // 4)
    idx_v[pl.ds(64, 16)] = jnp.where(pad, 0, s16)
    s16_v[...] = jnp.where(pad, n, s16)

    pltpu.async_copy(data_hbm.at[idx_v], rows_v, sem).wait()
    for k in range(4):
        pltpu.sync_copy(rows_v.at[pl.ds(16 * k, 16)],
                        g_hbm.at[pl.ds(512 * k + wid * 16, 16)])
    pltpu.sync_copy(rows_v.at[pl.ds(64, 16)],
                    g_hbm.at[pl.ds(4 * _SP + wid * 16, 16)])
    pltpu.sync_copy(val_v, valid_hbm.at[pl.ds(wid * 16, 16), :])
    pltpu.sync_copy(s16_v, srcs_hbm.at[pl.ds(wid * 16, 16)])


def _tc_dense(t_ref, data_ref, w1_ref, b1_ref, w2_ref, b2_ref, out_ref,
              a_scr, tvec_scr, *, c):
    j = pl.program_id(0)

    @pl.when(j == 0)
    def _():
        w1 = w1_ref[...]
        a_scr[...] = w1[1:1 + c, :]
        tvec_scr[...] = t_ref[0] * w1[0:1, :] + b1_ref[...].reshape(1, 15)

    pre = jnp.dot(data_ref[...], a_scr[...],
                  preferred_element_type=jnp.float32) + tvec_scr[...]
    out_ref[...] = jnp.dot(jnp.tanh(pre), w2_ref[...],
                           preferred_element_type=jnp.float32) \
        + b2_ref[...].reshape(1, c)


def _tc_fix(t_ref, srcs_ref, g_ref, val_ref, w1_ref, b1_ref, w2_ref, b2_ref,
            out0_ref, out_ref, rows_scr, sem, *, c, s_real):
    w1 = w1_ref[...]
    tvec = t_ref[0] * w1[0:1, :] + b1_ref[...].reshape(1, 15)
    gs = g_ref[4 * _SP:, :]
    pre = jnp.dot(gs, w1[1:1 + c, :],
                  preferred_element_type=jnp.float32) + tvec
    for k in range(4):
        gm = g_ref[_SP * k:_SP * (k + 1), :] * val_ref[:, k:k + 1]
        pre = pre + jnp.dot(gm, w1[1 + c * (k + 1):1 + c * (k + 2), :],
                            preferred_element_type=jnp.float32)
    rows_scr[...] = jnp.dot(jnp.tanh(pre), w2_ref[...],
                            preferred_element_type=jnp.float32) \
        + b2_ref[...].reshape(1, c)
    dmas = [
        pltpu.make_async_copy(rows_scr.at[pl.ds(s, 1)],
                              out_ref.at[pl.ds(srcs_ref[s], 1)], sem)
        for s in range(s_real)
    ]
    for d in dmas:
        d.start()
    for d in dmas:
        d.wait()


def kernel(t, data, edges, W1, b1, W2, b2):
    n, c = data.shape
    e = edges.shape[1]
    blk = 1000
    nblk = n // blk

    ef = edges.astype(jnp.int32).reshape(2 * e)
    tt = t.astype(jnp.float32)

    mesh = plsc.VectorSubcoreMesh(core_axis_name="c", subcore_axis_name="s")
    sc_gather = functools.partial(
        pl.kernel, mesh=mesh,
        compiler_params=pltpu.CompilerParams(needs_layout_passes=False),
        out_type=[
            jax.ShapeDtypeStruct((4 * _SP + _SP, c), jnp.float32),
            jax.ShapeDtypeStruct((_SP, 4), jnp.float32),
            jax.ShapeDtypeStruct((_SP,), jnp.int32),
        ],
        scratch_types=[
            pltpu.VMEM((2, 64), jnp.int32),
            pltpu.VMEM((16,), jnp.int32),
            pltpu.VMEM((16,), jnp.int32),
            pltpu.VMEM((80,), jnp.int32),
            pltpu.VMEM((16, 4), jnp.float32),
            pltpu.VMEM((80, c), jnp.float32),
            pltpu.SemaphoreType.DMA,
        ],
    )(functools.partial(_sc_gather, e=e, n=n))
    g, valid4, srcs = sc_gather(ef, data)
    g = jnp.zeros((4 * _SP + _SP, c), jnp.float32)
    valid4 = jnp.zeros((_SP, 4), jnp.float32)
    srcs = jnp.zeros((_SP,), jnp.int32)

    out0 = pl.pallas_call(
        functools.partial(_tc_dense, c=c),
        grid=(nblk,),
        in_specs=[
            pl.BlockSpec(memory_space=pltpu.SMEM),
            pl.BlockSpec((blk, c), lambda j: (j, 0)),
            pl.BlockSpec((641, 15), lambda j: (0, 0)),
            pl.BlockSpec((15,), lambda j: (0,)),
            pl.BlockSpec((15, c), lambda j: (0, 0)),
            pl.BlockSpec((c,), lambda j: (0,)),
        ],
        out_specs=pl.BlockSpec((blk, c), lambda j: (j, 0)),
        out_shape=jax.ShapeDtypeStruct((n, c), jnp.float32),
        scratch_shapes=[
            pltpu.VMEM((c, 15), jnp.float32),
            pltpu.VMEM((1, 15), jnp.float32),
        ],
    )(tt, data, W1, b1, W2, b2)

    out = pl.pallas_call(
        functools.partial(_tc_fix, c=c, s_real=e // 4),
        in_specs=[
            pl.BlockSpec(memory_space=pltpu.SMEM),
            pl.BlockSpec(memory_space=pltpu.SMEM),
            pl.BlockSpec((4 * _SP + _SP, c), lambda: (0, 0)),
            pl.BlockSpec((_SP, 4), lambda: (0, 0)),
            pl.BlockSpec((641, 15), lambda: (0, 0)),
            pl.BlockSpec((15,), lambda: (0,)),
            pl.BlockSpec((15, c), lambda: (0, 0)),
            pl.BlockSpec((c,), lambda: (0,)),
            pl.BlockSpec(memory_space=pl.ANY),
        ],
        out_specs=pl.BlockSpec(memory_space=pl.ANY),
        out_shape=jax.ShapeDtypeStruct((n, c), jnp.float32),
        input_output_aliases={8: 0},
        scratch_shapes=[
            pltpu.VMEM((_SP, c), jnp.float32),
            pltpu.SemaphoreType.DMA,
        ],
    )(tt, srcs, g, valid4, W1, b1, W2, b2, out0)
    return out


# speedup vs baseline: 1.0881x; 1.0881x over previous
import functools

import jax
import jax.numpy as jnp
from jax import lax
from jax.experimental import pallas as pl
from jax.experimental.pallas import tpu as pltpu
from jax.experimental.pallas import tpu_sc as plsc

_SENT = (1 << 14) - 1
_NC = 2
_NS = 16
_SP = 512


def _sc_gather(ef_hbm, data_hbm, g_hbm, valid_hbm, srcs_hbm,
               sd_v, s16_v, shift_v, idx_v, val_v, rows_v, sem,
               *, e, n):
    wid = lax.axis_index("s") * _NC + lax.axis_index("c")
    base = wid * 64
    ntile = (4 * _SP) // 64
    lane = jnp.arange(16, dtype=jnp.int32)
    row0 = jnp.zeros((16,), jnp.int32)
    row1 = jnp.full((16,), 1, jnp.int32)

    pltpu.sync_copy(ef_hbm.at[pl.ds(base, 64)], sd_v.at[0])

    @pl.when(wid < ntile - 1)
    def _():
        pltpu.sync_copy(ef_hbm.at[pl.ds(e + base, 64)], sd_v.at[1])

    @pl.when(wid == ntile - 1)
    def _():
        pltpu.sync_copy(ef_hbm.at[pl.ds(e + base, 16)],
                        sd_v.at[1, pl.ds(0, 16)])
        for v in range(1, 4):
            plsc.store_scatter(sd_v, [row1, 16 * v + lane],
                               jnp.full((16,), _SENT, jnp.int32))

    grp = (lane >> 2) << 14
    for v in range(4):
        d = plsc.load_gather(sd_v, [row1, 16 * v + lane])
        key = grp | d
        k1, _ = plsc.sort_key_val(key, lane)
        shift_v[...] = k1
        prev = plsc.load_gather(shift_v, [jnp.maximum(lane - 1, 0)])
        dup = (k1 == prev) & (lane != 0)
        k2 = jnp.where(dup, grp | _SENT, k1)
        k3, _ = plsc.sort_key_val(k2, lane)
        dstf = k3 & _SENT
        validb = dstf != _SENT
        posv = ((lane & 3) << 4) + 4 * v + (lane >> 2)
        plsc.store_scatter(idx_v, [posv], jnp.where(validb, dstf, 0))
        plsc.store_scatter(val_v, [4 * v + (lane >> 2), lane & 3],
                           validb.astype(jnp.float32))

    s16 = plsc.load_gather(sd_v, [row0, lane * 4])
    pad = (wid * 16 + lane) >= (e // 4)
    idx_v[pl.ds(64, 16)] = jnp.where(pad, 0, s16)
    s16_v[...] = jnp.where(pad, n, s16)

    pltpu.async_copy(data_hbm.at[idx_v], rows_v, sem).wait()
    for k in range(4):
        pltpu.sync_copy(rows_v.at[pl.ds(16 * k, 16)],
                        g_hbm.at[pl.ds(512 * k + wid * 16, 16)])
    pltpu.sync_copy(rows_v.at[pl.ds(64, 16)],
                    g_hbm.at[pl.ds(4 * _SP + wid * 16, 16)])
    pltpu.sync_copy(val_v, valid_hbm.at[pl.ds(wid * 16, 16), :])
    pltpu.sync_copy(s16_v, srcs_hbm.at[pl.ds(wid * 16, 16)])


def _tc_dense(t_ref, data_ref, w1_ref, b1_ref, w2_ref, b2_ref, out_ref,
              a_scr, tvec_scr, *, c):
    j = pl.program_id(0)

    @pl.when(j == 0)
    def _():
        w1 = w1_ref[...]
        a_scr[...] = w1[1:1 + c, :]
        tvec_scr[...] = t_ref[0] * w1[0:1, :] + b1_ref[...].reshape(1, 15)

    pre = jnp.dot(data_ref[...], a_scr[...],
                  preferred_element_type=jnp.float32) + tvec_scr[...]
    out_ref[...] = jnp.dot(jnp.tanh(pre), w2_ref[...],
                           preferred_element_type=jnp.float32) \
        + b2_ref[...].reshape(1, c)


def _tc_fix(t_ref, srcs_ref, g_ref, val_ref, w1_ref, b1_ref, w2_ref, b2_ref,
            out0_ref, out_ref, rows_scr, sem, *, c, s_real):
    w1 = w1_ref[...]
    tvec = t_ref[0] * w1[0:1, :] + b1_ref[...].reshape(1, 15)
    gs = g_ref[4 * _SP:, :]
    pre = jnp.dot(gs, w1[1:1 + c, :],
                  preferred_element_type=jnp.float32) + tvec
    for k in range(4):
        gm = g_ref[_SP * k:_SP * (k + 1), :] * val_ref[:, k:k + 1]
        pre = pre + jnp.dot(gm, w1[1 + c * (k + 1):1 + c * (k + 2), :],
                            preferred_element_type=jnp.float32)
    rows_scr[...] = jnp.dot(jnp.tanh(pre), w2_ref[...],
                            preferred_element_type=jnp.float32) \
        + b2_ref[...].reshape(1, c)
    dmas = [
        pltpu.make_async_copy(rows_scr.at[pl.ds(s, 1)],
                              out_ref.at[pl.ds(srcs_ref[s], 1)], sem)
        for s in range(s_real)
    ]
    for d in dmas:
        d.start()
    for d in dmas:
        d.wait()


def kernel(t, data, edges, W1, b1, W2, b2):
    n, c = data.shape
    e = edges.shape[1]
    blk = 2000
    nblk = n // blk

    ef = edges.astype(jnp.int32).reshape(2 * e)
    tt = t.astype(jnp.float32)

    mesh = plsc.VectorSubcoreMesh(core_axis_name="c", subcore_axis_name="s")
    sc_gather = functools.partial(
        pl.kernel, mesh=mesh,
        compiler_params=pltpu.CompilerParams(needs_layout_passes=False),
        out_type=[
            jax.ShapeDtypeStruct((4 * _SP + _SP, c), jnp.float32),
            jax.ShapeDtypeStruct((_SP, 4), jnp.float32),
            jax.ShapeDtypeStruct((_SP,), jnp.int32),
        ],
        scratch_types=[
            pltpu.VMEM((2, 64), jnp.int32),
            pltpu.VMEM((16,), jnp.int32),
            pltpu.VMEM((16,), jnp.int32),
            pltpu.VMEM((80,), jnp.int32),
            pltpu.VMEM((16, 4), jnp.float32),
            pltpu.VMEM((80, c), jnp.float32),
            pltpu.SemaphoreType.DMA,
        ],
    )(functools.partial(_sc_gather, e=e, n=n))
    g, valid4, srcs = sc_gather(ef, data)

    out0 = pl.pallas_call(
        functools.partial(_tc_dense, c=c),
        grid=(nblk,),
        in_specs=[
            pl.BlockSpec(memory_space=pltpu.SMEM),
            pl.BlockSpec((blk, c), lambda j: (j, 0)),
            pl.BlockSpec((641, 15), lambda j: (0, 0)),
            pl.BlockSpec((15,), lambda j: (0,)),
            pl.BlockSpec((15, c), lambda j: (0, 0)),
            pl.BlockSpec((c,), lambda j: (0,)),
        ],
        out_specs=pl.BlockSpec((blk, c), lambda j: (j, 0)),
        out_shape=jax.ShapeDtypeStruct((n, c), jnp.float32),
        scratch_shapes=[
            pltpu.VMEM((c, 15), jnp.float32),
            pltpu.VMEM((1, 15), jnp.float32),
        ],
    )(tt, data, W1, b1, W2, b2)

    out = pl.pallas_call(
        functools.partial(_tc_fix, c=c, s_real=e // 4),
        in_specs=[
            pl.BlockSpec(memory_space=pltpu.SMEM),
            pl.BlockSpec(memory_space=pltpu.SMEM),
            pl.BlockSpec((4 * _SP + _SP, c), lambda: (0, 0)),
            pl.BlockSpec((_SP, 4), lambda: (0, 0)),
            pl.BlockSpec((641, 15), lambda: (0, 0)),
            pl.BlockSpec((15,), lambda: (0,)),
            pl.BlockSpec((15, c), lambda: (0, 0)),
            pl.BlockSpec((c,), lambda: (0,)),
            pl.BlockSpec(memory_space=pl.ANY),
        ],
        out_specs=pl.BlockSpec(memory_space=pl.ANY),
        out_shape=jax.ShapeDtypeStruct((n, c), jnp.float32),
        input_output_aliases={8: 0},
        scratch_shapes=[
            pltpu.VMEM((_SP, c), jnp.float32),
            pltpu.SemaphoreType.DMA,
        ],
    )(tt, srcs, g, valid4, W1, b1, W2, b2, out0)
    return out


# speedup vs baseline: 1.1016x; 1.0124x over previous
import functools

import jax
import jax.numpy as jnp
from jax import lax
from jax.experimental import pallas as pl
from jax.experimental.pallas import tpu as pltpu
from jax.experimental.pallas import tpu_sc as plsc

_SENT = (1 << 14) - 1
_NC = 2
_NS = 16
_SP = 512


def _sc_gather(ef_hbm, data_hbm, g_hbm, valid_hbm,
               sd_v, shift_v, idx_v, val_v, rows_v, sem, *, e, n):
    wid = lax.axis_index("s") * _NC + lax.axis_index("c")
    base = wid * 64
    ntile = (4 * _SP) // 64
    lane = jnp.arange(16, dtype=jnp.int32)
    row0 = jnp.zeros((16,), jnp.int32)
    row1 = jnp.full((16,), 1, jnp.int32)

    pltpu.sync_copy(ef_hbm.at[pl.ds(base, 64)], sd_v.at[0])

    @pl.when(wid < ntile - 1)
    def _():
        pltpu.sync_copy(ef_hbm.at[pl.ds(e + base, 64)], sd_v.at[1])

    @pl.when(wid == ntile - 1)
    def _():
        pltpu.sync_copy(ef_hbm.at[pl.ds(e + base, 16)],
                        sd_v.at[1, pl.ds(0, 16)])
        for v in range(1, 4):
            plsc.store_scatter(sd_v, [row1, 16 * v + lane],
                               jnp.full((16,), _SENT, jnp.int32))

    grp = (lane >> 2) << 14
    for v in range(4):
        d = plsc.load_gather(sd_v, [row1, 16 * v + lane])
        key = grp | d
        k1, _ = plsc.sort_key_val(key, lane)
        shift_v[...] = k1
        prev = plsc.load_gather(shift_v, [jnp.maximum(lane - 1, 0)])
        dup = (k1 == prev) & (lane != 0)
        k2 = jnp.where(dup, grp | _SENT, k1)
        k3, _ = plsc.sort_key_val(k2, lane)
        dstf = k3 & _SENT
        validb = dstf != _SENT
        posv = ((lane & 3) << 4) + 4 * v + (lane >> 2)
        plsc.store_scatter(idx_v, [posv], jnp.where(validb, dstf, 0))
        plsc.store_scatter(val_v, [4 * v + (lane >> 2), lane & 3],
                           validb.astype(jnp.float32))

    s16 = plsc.load_gather(sd_v, [row0, lane * 4])
    pad = (wid * 16 + lane) >= (e // 4)
    idx_v[pl.ds(64, 16)] = jnp.where(pad, 0, s16)

    pltpu.async_copy(data_hbm.at[idx_v], rows_v, sem).wait()
    for k in range(4):
        pltpu.sync_copy(rows_v.at[pl.ds(16 * k, 16)],
                        g_hbm.at[pl.ds(512 * k + wid * 16, 16)])
    pltpu.sync_copy(rows_v.at[pl.ds(64, 16)],
                    g_hbm.at[pl.ds(4 * _SP + wid * 16, 16)])
    pltpu.sync_copy(val_v, valid_hbm.at[pl.ds(wid * 16, 16), :])


def _tc_dense(t_ref, data_ref, w1_ref, b1_ref, w2_ref, b2_ref, out_ref,
              a_scr, tvec_scr, *, c):
    j = pl.program_id(0)

    @pl.when(j == 0)
    def _():
        w1 = w1_ref[...]
        a_scr[...] = w1[1:1 + c, :]
        tvec_scr[...] = t_ref[0] * w1[0:1, :] + b1_ref[...].reshape(1, 15)

    pre = jnp.dot(data_ref[...], a_scr[...],
                  preferred_element_type=jnp.float32) + tvec_scr[...]
    out_ref[...] = jnp.dot(jnp.tanh(pre), w2_ref[...],
                           preferred_element_type=jnp.float32) \
        + b2_ref[...].reshape(1, c)


def _tc_fix(t_ref, srcs_ref, g_ref, val_ref, w1_ref, b1_ref, w2_ref, b2_ref,
            out0_ref, out_ref, rows_scr, sem, *, c, s_real):
    w1 = w1_ref[...]
    tvec = t_ref[0] * w1[0:1, :] + b1_ref[...].reshape(1, 15)
    gs = g_ref[4 * _SP:, :]
    pre = jnp.dot(gs, w1[1:1 + c, :],
                  preferred_element_type=jnp.float32) + tvec
    for k in range(4):
        gm = g_ref[_SP * k:_SP * (k + 1), :] * val_ref[:, k:k + 1]
        pre = pre + jnp.dot(gm, w1[1 + c * (k + 1):1 + c * (k + 2), :],
                            preferred_element_type=jnp.float32)
    rows_scr[...] = jnp.dot(jnp.tanh(pre), w2_ref[...],
                            preferred_element_type=jnp.float32) \
        + b2_ref[...].reshape(1, c)
    dmas = [
        pltpu.make_async_copy(rows_scr.at[pl.ds(s, 1)],
                              out_ref.at[pl.ds(srcs_ref[4 * s], 1)], sem)
        for s in range(s_real)
    ]
    for d in dmas:
        d.start()
    for d in dmas:
        d.wait()


def kernel(t, data, edges, W1, b1, W2, b2):
    n, c = data.shape
    e = edges.shape[1]
    blk = 2000
    nblk = n // blk

    ef = edges.astype(jnp.int32).reshape(2 * e)
    tt = t.astype(jnp.float32)

    mesh = plsc.VectorSubcoreMesh(core_axis_name="c", subcore_axis_name="s")
    sc_gather = functools.partial(
        pl.kernel, mesh=mesh,
        compiler_params=pltpu.CompilerParams(needs_layout_passes=False),
        out_type=[
            jax.ShapeDtypeStruct((4 * _SP + _SP, c), jnp.float32),
            jax.ShapeDtypeStruct((_SP, 4), jnp.float32),
        ],
        scratch_types=[
            pltpu.VMEM((2, 64), jnp.int32),
            pltpu.VMEM((16,), jnp.int32),
            pltpu.VMEM((80,), jnp.int32),
            pltpu.VMEM((16, 4), jnp.float32),
            pltpu.VMEM((80, c), jnp.float32),
            pltpu.SemaphoreType.DMA,
        ],
    )(functools.partial(_sc_gather, e=e, n=n))
    g, valid4 = sc_gather(ef, data)

    out0 = pl.pallas_call(
        functools.partial(_tc_dense, c=c),
        grid=(nblk,),
        in_specs=[
            pl.BlockSpec(memory_space=pltpu.SMEM),
            pl.BlockSpec((blk, c), lambda j: (j, 0)),
            pl.BlockSpec((641, 15), lambda j: (0, 0)),
            pl.BlockSpec((15,), lambda j: (0,)),
            pl.BlockSpec((15, c), lambda j: (0, 0)),
            pl.BlockSpec((c,), lambda j: (0,)),
        ],
        out_specs=pl.BlockSpec((blk, c), lambda j: (j, 0)),
        out_shape=jax.ShapeDtypeStruct((n, c), jnp.float32),
        scratch_shapes=[
            pltpu.VMEM((c, 15), jnp.float32),
            pltpu.VMEM((1, 15), jnp.float32),
        ],
    )(tt, data, W1, b1, W2, b2)

    out = pl.pallas_call(
        functools.partial(_tc_fix, c=c, s_real=e // 4),
        in_specs=[
            pl.BlockSpec(memory_space=pltpu.SMEM),
            pl.BlockSpec(memory_space=pltpu.SMEM),
            pl.BlockSpec((4 * _SP + _SP, c), lambda: (0, 0)),
            pl.BlockSpec((_SP, 4), lambda: (0, 0)),
            pl.BlockSpec((641, 15), lambda: (0, 0)),
            pl.BlockSpec((15,), lambda: (0,)),
            pl.BlockSpec((15, c), lambda: (0, 0)),
            pl.BlockSpec((c,), lambda: (0,)),
            pl.BlockSpec(memory_space=pl.ANY),
        ],
        out_specs=pl.BlockSpec(memory_space=pl.ANY),
        out_shape=jax.ShapeDtypeStruct((n, c), jnp.float32),
        input_output_aliases={8: 0},
        scratch_shapes=[
            pltpu.VMEM((_SP, c), jnp.float32),
            pltpu.SemaphoreType.DMA,
        ],
    )(tt, ef, g, valid4, W1, b1, W2, b2, out0)
    return out
